# Initial kernel scaffold; baseline (speedup 1.0000x reference)
import functools

import jax
import jax.numpy as jnp
from jax import lax
from jax.experimental import pallas as pl
from jax.experimental.pallas import tpu as pltpu
from jax.experimental.pallas import tpu_sc as plsc

SCALE = 0.25
POOLED = 7
BINS = POOLED * POOLED
SAMPLES = BINS * 4
CHUNK = POOLED * 16
NCORES = 2
NSUB = 16
NWORKERS = NCORES * NSUB
L = 16


def _splat(v):
    return jnp.full((L,), v, dtype=jnp.int32)


def _roi_align_sc(flat, params, npad, H, W, C):
    rpw = npad // NWORKERS
    ncb = C // L
    mesh = plsc.VectorSubcoreMesh(
        core_axis_name="c", subcore_axis_name="s",
        num_cores=NCORES, num_subcores=NSUB)

    @functools.partial(
        pl.kernel,
        out_type=jax.ShapeDtypeStruct((npad, C, BINS), jnp.float32),
        mesh=mesh,
        scratch_types=[
            pltpu.VMEM((rpw, L), jnp.float32),
            pltpu.VMEM((POOLED, CHUNK), jnp.int32),
            pltpu.VMEM((POOLED, CHUNK), jnp.float32),
            pltpu.VMEM((2, CHUNK, C), jnp.float32),
            pltpu.VMEM((C, BINS), jnp.float32),
            pltpu.SemaphoreType.DMA,
            pltpu.SemaphoreType.DMA,
        ],
    )
    def k(flat_hbm, params_hbm, out_hbm,
          params_v, idx_v, w_v, rows_v, outT_v, sem0, sem1):
        sems = (sem0, sem1)
        wid = lax.axis_index("s") * NCORES + lax.axis_index("c")
        pltpu.sync_copy(params_hbm.at[pl.ds(wid * rpw, rpw)], params_v)
        lanes = lax.iota(jnp.int32, (L,), 0)

        @pl.loop(0, rpw)
        def _roi(i):
            iv = jnp.full((L,), i, dtype=jnp.int32)

            def pget(j):
                return plsc.load_gather(params_v, [iv, _splat(j)])

            base_i = pget(0).astype(jnp.int32)
            cxv, cyv = pget(1), pget(2)
            rwv, rhv = pget(3), pget(4)
            csv, snv = pget(5), pget(6)

            for j in range((SAMPLES + L - 1) // L):
                s = lanes + (L * j)
                ph = lax.div(s, _splat(28))
                r = lax.rem(s, _splat(28))
                pw = lax.div(r, _splat(4))
                q = lax.rem(r, _splat(4))
                iy = lax.div(q, _splat(2))
                ix = lax.rem(q, _splat(2))
                fy = (ph.astype(jnp.float32)
                      + (iy.astype(jnp.float32) * 0.5 + 0.25)) * (1.0 / POOLED) - 0.5
                fx = (pw.astype(jnp.float32)
                      + (ix.astype(jnp.float32) * 0.5 + 0.25)) * (1.0 / POOLED) - 0.5
                yy = rhv * fy
                xx = rwv * fx
                y = yy * csv - xx * snv + cyv
                x = yy * snv + xx * csv + cxv
                valid = (y > -1.0) & (y < H) & (x > -1.0) & (x < W)
                yc = jnp.maximum(y, 0.0)
                xc = jnp.maximum(x, 0.0)
                yl = yc.astype(jnp.int32)
                xl = xc.astype(jnp.int32)
                ycond = yl >= H - 1
                xcond = xl >= W - 1
                yl = jnp.where(ycond, H - 1, yl)
                xl = jnp.where(xcond, W - 1, xl)
                yh = jnp.where(ycond, H - 1, yl + 1)
                xh = jnp.where(xcond, W - 1, xl + 1)
                ly = jnp.where(ycond, 0.0, yc - yl.astype(jnp.float32))
                lx = jnp.where(xcond, 0.0, xc - xl.astype(jnp.float32))
                hy = 1.0 - ly
                hx = 1.0 - lx
                vm = jnp.where(valid, 0.25, 0.0)
                rl = base_i + yl * W
                rh_ = base_i + yh * W
                col = r * 4
                mask = s < SAMPLES
                entries = (
                    (rl + xl, hy * hx * vm),
                    (rl + xh, hy * lx * vm),
                    (rh_ + xl, ly * hx * vm),
                    (rh_ + xh, ly * lx * vm),
                )
                for c, (ivec, wvec) in enumerate(entries):
                    plsc.store_scatter(idx_v, [ph, col + c], ivec, mask=mask)
                    plsc.store_scatter(w_v, [ph, col + c], wvec, mask=mask)

            def start(ph):
                slot = ph % 2
                return pltpu.async_copy(
                    flat_hbm.at[idx_v.at[ph]], rows_v.at[slot], sems[slot])

            handle = start(0)
            for ph in range(POOLED):
                nxt = start(ph + 1) if ph + 1 < POOLED else None
                handle.wait()
                slot = ph % 2
                ph_full = _splat(ph)

                @pl.loop(0, POOLED)
                def _bin(pw, ph=ph, ph_full=ph_full, slot=slot):
                    accs = [None] * ncb
                    for kk in range(16):
                        wv = plsc.load_gather(
                            w_v, [ph_full, jnp.full((L,), pw * 16 + kk, jnp.int32)])
                        row = pw * 16 + kk
                        for cb in range(ncb):
                            blk = rows_v[slot, row, pl.ds(cb * L, L)]
                            accs[cb] = wv * blk if kk == 0 else accs[cb] + wv * blk
                    bcol = jnp.full((L,), ph * POOLED + pw, dtype=jnp.int32)
                    for cb in range(ncb):
                        plsc.store_scatter(outT_v, [lanes + cb * L, bcol], accs[cb])

                handle = nxt
            pltpu.sync_copy(outT_v, out_hbm.at[wid * rpw + i])

    return k(flat, params)


def kernel(input, rois):
    B, C, H, W = input.shape
    N = rois.shape[0]
    npad = -(-N // NWORKERS) * NWORKERS
    flat = input.transpose(0, 2, 3, 1).reshape(B * H * W, C)
    batch = rois[:, 0].astype(jnp.int32)
    base = (batch * (H * W)).astype(jnp.float32)
    cx = rois[:, 1] * SCALE - 0.5
    cy = rois[:, 2] * SCALE - 0.5
    rw = rois[:, 3] * SCALE
    rh = rois[:, 4] * SCALE
    th = rois[:, 5]
    params = jnp.stack(
        [base, cx, cy, rw, rh, jnp.cos(th), jnp.sin(th)], axis=1)
    params = jnp.pad(params, ((0, npad - N), (0, L - params.shape[1])))
    out = _roi_align_sc(flat, params, npad, H, W, C)
    return out[:N].reshape(N, C, POOLED, POOLED)


# speedup vs baseline: 5.1909x; 5.1909x over previous
import functools

import jax
import jax.numpy as jnp
from jax import lax
from jax.experimental import pallas as pl
from jax.experimental.pallas import tpu as pltpu
from jax.experimental.pallas import tpu_sc as plsc

SCALE = 0.25
POOLED = 7
BINS = POOLED * POOLED
SAMPLES = BINS * 4
CHUNK = POOLED * 16
NCORES = 2
NSUB = 16
NWORKERS = NCORES * NSUB
L = 16


def _splat(v):
    return jnp.full((L,), v, dtype=jnp.int32)


def _roi_align_sc(flat, params, npad, H, W, C):
    rpw = npad // NWORKERS
    ncb = C // L
    mesh = plsc.VectorSubcoreMesh(
        core_axis_name="c", subcore_axis_name="s",
        num_cores=NCORES, num_subcores=NSUB)

    @functools.partial(
        pl.kernel,
        out_type=jax.ShapeDtypeStruct((npad, C * BINS), jnp.float32),
        mesh=mesh,
        scratch_types=[
            pltpu.VMEM((rpw * L,), jnp.float32),
            pltpu.VMEM((POOLED * CHUNK,), jnp.int32),
            pltpu.VMEM((POOLED * CHUNK,), jnp.float32),
            pltpu.VMEM((2, CHUNK, C), jnp.float32),
            pltpu.VMEM((C * BINS,), jnp.float32),
            pltpu.SemaphoreType.DMA,
            pltpu.SemaphoreType.DMA,
        ],
        compiler_params=pltpu.CompilerParams(needs_layout_passes=False),
    )
    def k(flat_hbm, params_hbm, out_hbm,
          params_v, idx_v, w_v, rows_v, outT_v, sem0, sem1):
        sems = (sem0, sem1)
        wid = lax.axis_index("s") * NCORES + lax.axis_index("c")
        pltpu.sync_copy(params_hbm.at[pl.ds(wid * (rpw * L), rpw * L)], params_v)
        lanes = lax.iota(jnp.int32, L)

        @pl.loop(0, rpw)
        def _roi(i):
            iv = jnp.full((L,), i, dtype=jnp.int32)

            def pget(j):
                return plsc.load_gather(params_v, [iv * L + _splat(j)])

            base_i = pget(0).astype(jnp.int32)
            cxv, cyv = pget(1), pget(2)
            rwv, rhv = pget(3), pget(4)
            csv, snv = pget(5), pget(6)

            for j in range((SAMPLES + L - 1) // L):
                s = lanes + (L * j)
                ph = lax.div(s, _splat(28))
                r = lax.rem(s, _splat(28))
                pw = lax.div(r, _splat(4))
                q = lax.rem(r, _splat(4))
                iy = lax.div(q, _splat(2))
                ix = lax.rem(q, _splat(2))
                fy = (ph.astype(jnp.float32)
                      + (iy.astype(jnp.float32) * 0.5 + 0.25)) * (1.0 / POOLED) - 0.5
                fx = (pw.astype(jnp.float32)
                      + (ix.astype(jnp.float32) * 0.5 + 0.25)) * (1.0 / POOLED) - 0.5
                yy = rhv * fy
                xx = rwv * fx
                y = yy * csv - xx * snv + cyv
                x = yy * snv + xx * csv + cxv
                valid = (y > -1.0) & (y < H) & (x > -1.0) & (x < W)
                yc = jnp.maximum(y, 0.0)
                xc = jnp.maximum(x, 0.0)
                yl = yc.astype(jnp.int32)
                xl = xc.astype(jnp.int32)
                ycond = yl >= H - 1
                xcond = xl >= W - 1
                yl = jnp.where(ycond, H - 1, yl)
                xl = jnp.where(xcond, W - 1, xl)
                yh = jnp.where(ycond, H - 1, yl + 1)
                xh = jnp.where(xcond, W - 1, xl + 1)
                ly = jnp.where(ycond, 0.0, yc - yl.astype(jnp.float32))
                lx = jnp.where(xcond, 0.0, xc - xl.astype(jnp.float32))
                hy = 1.0 - ly
                hx = 1.0 - lx
                vm = jnp.where(valid, 0.25, 0.0)
                rl = base_i + yl * W
                rh_ = base_i + yh * W
                col = s * 4
                mask = s < SAMPLES
                entries = (
                    (rl + xl, hy * hx * vm),
                    (rl + xh, hy * lx * vm),
                    (rh_ + xl, ly * hx * vm),
                    (rh_ + xh, ly * lx * vm),
                )
                for c, (ivec, wvec) in enumerate(entries):
                    plsc.store_scatter(idx_v, [col + c], ivec, mask=mask)
                    plsc.store_scatter(w_v, [col + c], wvec, mask=mask)

            def start(ph):
                slot = ph % 2
                return pltpu.async_copy(
                    flat_hbm.at[idx_v.at[pl.ds(ph * CHUNK, CHUNK)]],
                    rows_v.at[slot], sems[slot])

            handle = start(0)
            for ph in range(POOLED):
                nxt = start(ph + 1) if ph + 1 < POOLED else None
                handle.wait()
                slot = ph % 2

                @pl.loop(0, POOLED)
                def _bin(pw, ph=ph, slot=slot):
                    accs = [None] * ncb
                    for kk in range(16):
                        wv = plsc.load_gather(
                            w_v, [jnp.full((L,), ph * CHUNK + pw * 16 + kk,
                                           jnp.int32)])
                        row = pw * 16 + kk
                        for cb in range(ncb):
                            blk = rows_v[slot, row, pl.ds(cb * L, L)]
                            accs[cb] = wv * blk if kk == 0 else accs[cb] + wv * blk
                    bcol = jnp.full((L,), ph * POOLED + pw, dtype=jnp.int32)
                    for cb in range(ncb):
                        plsc.store_scatter(
                            outT_v, [(lanes + cb * L) * BINS + bcol], accs[cb])

                handle = nxt
            pltpu.sync_copy(outT_v, out_hbm.at[wid * rpw + i])

    return k(flat, params)


def kernel(input, rois):
    B, C, H, W = input.shape
    N = rois.shape[0]
    npad = -(-N // NWORKERS) * NWORKERS
    flat = input.transpose(0, 2, 3, 1).reshape(B * H * W, C)
    batch = rois[:, 0].astype(jnp.int32)
    base = (batch * (H * W)).astype(jnp.float32)
    cx = rois[:, 1] * SCALE - 0.5
    cy = rois[:, 2] * SCALE - 0.5
    rw = rois[:, 3] * SCALE
    rh = rois[:, 4] * SCALE
    th = rois[:, 5]
    params = jnp.stack(
        [base, cx, cy, rw, rh, jnp.cos(th), jnp.sin(th)], axis=1)
    params = jnp.pad(params, ((0, npad - N), (0, L - params.shape[1])))
    params = params.reshape(npad * L)
    out = _roi_align_sc(flat, params, npad, H, W, C)
    return out[:N].reshape(N, C, POOLED, POOLED)


# speedup vs baseline: 5.9655x; 1.1492x over previous
import functools

import jax
import jax.numpy as jnp
from jax import lax
from jax.experimental import pallas as pl
from jax.experimental.pallas import tpu as pltpu
from jax.experimental.pallas import tpu_sc as plsc

SCALE = 0.25
POOLED = 7
BINS = POOLED * POOLED
SAMPLES = BINS * 4
CHUNK = POOLED * 16
NCORES = 2
NSUB = 16
NWORKERS = NCORES * NSUB
L = 16


def _splat(v):
    return jnp.full((L,), v, dtype=jnp.int32)


def _roi_align_sc(flat, params, npad, N, H, W, C):
    rpw = npad // NWORKERS
    ncb = C // L
    mesh = plsc.VectorSubcoreMesh(
        core_axis_name="c", subcore_axis_name="s",
        num_cores=NCORES, num_subcores=NSUB)

    @functools.partial(
        pl.kernel,
        out_type=jax.ShapeDtypeStruct((N, C * BINS), jnp.float32),
        mesh=mesh,
        scratch_types=[
            pltpu.VMEM((rpw * L,), jnp.float32),
            pltpu.VMEM((POOLED * CHUNK,), jnp.int32),
            pltpu.VMEM((POOLED * CHUNK,), jnp.float32),
            pltpu.VMEM((2, CHUNK, C), jnp.float32),
            pltpu.VMEM((C * BINS,), jnp.float32),
            pltpu.SemaphoreType.DMA,
            pltpu.SemaphoreType.DMA,
        ],
        compiler_params=pltpu.CompilerParams(needs_layout_passes=False),
    )
    def k(flat_hbm, params_hbm, out_hbm,
          params_v, idx_v, w_v, rows_v, outT_v, sem0, sem1):
        sems = (sem0, sem1)
        wid = lax.axis_index("s") * NCORES + lax.axis_index("c")
        pltpu.sync_copy(params_hbm.at[pl.ds(wid * (rpw * L), rpw * L)], params_v)
        lanes = lax.iota(jnp.int32, L)

        def _roi_body(i, roi):
            iv = jnp.full((L,), i, dtype=jnp.int32)

            def pget(j):
                return plsc.load_gather(params_v, [iv * L + _splat(j)])

            base_i = pget(0).astype(jnp.int32)
            cxv, cyv = pget(1), pget(2)
            rwv, rhv = pget(3), pget(4)
            csv, snv = pget(5), pget(6)

            for j in range((SAMPLES + L - 1) // L):
                s = lanes + (L * j)
                ph = lax.div(s, _splat(28))
                r = lax.rem(s, _splat(28))
                pw = lax.div(r, _splat(4))
                q = lax.rem(r, _splat(4))
                iy = lax.div(q, _splat(2))
                ix = lax.rem(q, _splat(2))
                fy = (ph.astype(jnp.float32)
                      + (iy.astype(jnp.float32) * 0.5 + 0.25)) * (1.0 / POOLED) - 0.5
                fx = (pw.astype(jnp.float32)
                      + (ix.astype(jnp.float32) * 0.5 + 0.25)) * (1.0 / POOLED) - 0.5
                yy = rhv * fy
                xx = rwv * fx
                y = yy * csv - xx * snv + cyv
                x = yy * snv + xx * csv + cxv
                valid = (y > -1.0) & (y < H) & (x > -1.0) & (x < W)
                yc = jnp.maximum(y, 0.0)
                xc = jnp.maximum(x, 0.0)
                yl = yc.astype(jnp.int32)
                xl = xc.astype(jnp.int32)
                ycond = yl >= H - 1
                xcond = xl >= W - 1
                yl = jnp.where(ycond, H - 1, yl)
                xl = jnp.where(xcond, W - 1, xl)
                yh = jnp.where(ycond, H - 1, yl + 1)
                xh = jnp.where(xcond, W - 1, xl + 1)
                ly = jnp.where(ycond, 0.0, yc - yl.astype(jnp.float32))
                lx = jnp.where(xcond, 0.0, xc - xl.astype(jnp.float32))
                hy = 1.0 - ly
                hx = 1.0 - lx
                vm = jnp.where(valid, 0.25, 0.0)
                rl = base_i + yl * W
                rh_ = base_i + yh * W
                col = s * 4
                mask = s < SAMPLES
                entries = (
                    (rl + xl, hy * hx * vm),
                    (rl + xh, hy * lx * vm),
                    (rh_ + xl, ly * hx * vm),
                    (rh_ + xh, ly * lx * vm),
                )
                for c, (ivec, wvec) in enumerate(entries):
                    plsc.store_scatter(idx_v, [col + c], ivec, mask=mask)
                    plsc.store_scatter(w_v, [col + c], wvec, mask=mask)

            def start(ph):
                slot = ph % 2
                return pltpu.async_copy(
                    flat_hbm.at[idx_v.at[pl.ds(ph * CHUNK, CHUNK)]],
                    rows_v.at[slot], sems[slot])

            handle = start(0)
            for ph in range(POOLED):
                nxt = start(ph + 1) if ph + 1 < POOLED else None
                handle.wait()
                slot = ph % 2

                @pl.loop(0, POOLED)
                def _bin(pw, ph=ph, slot=slot):
                    accs = [None] * ncb
                    for kk in range(16):
                        wv = plsc.load_gather(
                            w_v, [jnp.full((L,), ph * CHUNK + pw * 16 + kk,
                                           jnp.int32)])
                        row = pw * 16 + kk
                        for cb in range(ncb):
                            blk = rows_v[slot, row, pl.ds(cb * L, L)]
                            accs[cb] = wv * blk if kk == 0 else accs[cb] + wv * blk
                    bcol = jnp.full((L,), ph * POOLED + pw, dtype=jnp.int32)
                    for cb in range(ncb):
                        plsc.store_scatter(
                            outT_v, [(lanes + cb * L) * BINS + bcol], accs[cb])

                handle = nxt
            pltpu.sync_copy(outT_v, out_hbm.at[roi])

        @pl.loop(0, rpw)
        def _roi_guard(i):
            roi = wid * rpw + i

            @pl.when(roi < N)
            def _roi():
                _roi_body(i, roi)

    return k(flat, params)


def kernel(input, rois):
    B, C, H, W = input.shape
    N = rois.shape[0]
    npad = -(-N // NWORKERS) * NWORKERS
    flat = input.transpose(0, 2, 3, 1).reshape(B * H * W, C)
    batch = rois[:, 0].astype(jnp.int32)
    base = (batch * (H * W)).astype(jnp.float32)
    cx = rois[:, 1] * SCALE - 0.5
    cy = rois[:, 2] * SCALE - 0.5
    rw = rois[:, 3] * SCALE
    rh = rois[:, 4] * SCALE
    th = rois[:, 5]
    params = jnp.stack(
        [base, cx, cy, rw, rh, jnp.cos(th), jnp.sin(th)], axis=1)
    params = jnp.pad(params, ((0, npad - N), (0, L - params.shape[1])))
    params = params.reshape(npad * L)
    out = _roi_align_sc(flat, params, npad, N, H, W, C)
    return out.reshape(N, C, POOLED, POOLED)


# speedup vs baseline: 6.5997x; 1.1063x over previous
import functools

import jax
import jax.numpy as jnp
from jax import lax
from jax.experimental import pallas as pl
from jax.experimental.pallas import tpu as pltpu
from jax.experimental.pallas import tpu_sc as plsc

SCALE = 0.25
POOLED = 7
BINS = POOLED * POOLED
SAMPLES = BINS * 4
CHUNK = POOLED * 16
NCORES = 2
NSUB = 16
NWORKERS = NCORES * NSUB
L = 16


def _splat(v):
    return jnp.full((L,), v, dtype=jnp.int32)


def _roi_align_sc(flat, params, npad, N, H, W, C):
    rpw = npad // NWORKERS
    ng = C // (2 * L)
    mesh = plsc.VectorSubcoreMesh(
        core_axis_name="c", subcore_axis_name="s",
        num_cores=NCORES, num_subcores=NSUB)

    @functools.partial(
        pl.kernel,
        out_type=jax.ShapeDtypeStruct((N, C * BINS), jnp.float32),
        mesh=mesh,
        scratch_types=[
            pltpu.VMEM((rpw * L,), jnp.float32),
            pltpu.VMEM((POOLED * CHUNK,), jnp.int32),
            pltpu.VMEM((POOLED * CHUNK,), jnp.float32),
            pltpu.VMEM((2, CHUNK, C // 2), jnp.int32),
            pltpu.VMEM((C * BINS,), jnp.float32),
            pltpu.SemaphoreType.DMA,
            pltpu.SemaphoreType.DMA,
        ],
        compiler_params=pltpu.CompilerParams(needs_layout_passes=False),
    )
    def k(flat_hbm, params_hbm, out_hbm,
          params_v, idx_v, w_v, rows_v, outT_v, sem0, sem1):
        sems = (sem0, sem1)
        wid = lax.axis_index("s") * NCORES + lax.axis_index("c")
        pltpu.sync_copy(params_hbm.at[pl.ds(wid * (rpw * L), rpw * L)], params_v)
        lanes = lax.iota(jnp.int32, L)

        def _roi_body(i, roi):
            iv = jnp.full((L,), i, dtype=jnp.int32)

            def pget(j):
                return plsc.load_gather(params_v, [iv * L + _splat(j)])

            base_i = pget(0).astype(jnp.int32)
            cxv, cyv = pget(1), pget(2)
            rwv, rhv = pget(3), pget(4)
            csv, snv = pget(5), pget(6)

            for j in range((SAMPLES + L - 1) // L):
                s = lanes + (L * j)
                ph = lax.div(s, _splat(28))
                r = lax.rem(s, _splat(28))
                pw = lax.div(r, _splat(4))
                q = lax.rem(r, _splat(4))
                iy = lax.div(q, _splat(2))
                ix = lax.rem(q, _splat(2))
                fy = (ph.astype(jnp.float32)
                      + (iy.astype(jnp.float32) * 0.5 + 0.25)) * (1.0 / POOLED) - 0.5
                fx = (pw.astype(jnp.float32)
                      + (ix.astype(jnp.float32) * 0.5 + 0.25)) * (1.0 / POOLED) - 0.5
                yy = rhv * fy
                xx = rwv * fx
                y = yy * csv - xx * snv + cyv
                x = yy * snv + xx * csv + cxv
                valid = (y > -1.0) & (y < H) & (x > -1.0) & (x < W)
                yc = jnp.maximum(y, 0.0)
                xc = jnp.maximum(x, 0.0)
                yl = yc.astype(jnp.int32)
                xl = xc.astype(jnp.int32)
                ycond = yl >= H - 1
                xcond = xl >= W - 1
                yl = jnp.where(ycond, H - 1, yl)
                xl = jnp.where(xcond, W - 1, xl)
                yh = jnp.where(ycond, H - 1, yl + 1)
                xh = jnp.where(xcond, W - 1, xl + 1)
                ly = jnp.where(ycond, 0.0, yc - yl.astype(jnp.float32))
                lx = jnp.where(xcond, 0.0, xc - xl.astype(jnp.float32))
                hy = 1.0 - ly
                hx = 1.0 - lx
                vm = jnp.where(valid, 0.25, 0.0)
                rl = base_i + yl * W
                rh_ = base_i + yh * W
                col = s * 4
                mask = s < SAMPLES
                entries = (
                    (rl + xl, hy * hx * vm),
                    (rl + xh, hy * lx * vm),
                    (rh_ + xl, ly * hx * vm),
                    (rh_ + xh, ly * lx * vm),
                )
                for c, (ivec, wvec) in enumerate(entries):
                    plsc.store_scatter(idx_v, [col + c], ivec, mask=mask)
                    plsc.store_scatter(w_v, [col + c], wvec, mask=mask)

            def start(ph):
                slot = ph % 2
                return pltpu.async_copy(
                    flat_hbm.at[idx_v.at[pl.ds(ph * CHUNK, CHUNK)]],
                    rows_v.at[slot], sems[slot])

            handle = start(0)
            for ph in range(POOLED):
                nxt = start(ph + 1) if ph + 1 < POOLED else None
                handle.wait()
                slot = ph % 2

                @pl.loop(0, POOLED)
                def _bin(pw, ph=ph, slot=slot):
                    acca = [None] * ng
                    accb = [None] * ng
                    for kk in range(16):
                        wv = plsc.load_gather(
                            w_v, [jnp.full((L,), ph * CHUNK + pw * 16 + kk,
                                           jnp.int32)])
                        row = pw * 16 + kk
                        for g in range(ng):
                            blk = plsc.bitcast(
                                rows_v[slot, row, pl.ds(g * L, L)], jnp.bfloat16)
                            a, b = plsc.unpack(
                                blk, format=plsc.PackFormat.INTERLEAVED)
                            if kk == 0:
                                acca[g] = wv * a
                                accb[g] = wv * b
                            else:
                                acca[g] = acca[g] + wv * a
                                accb[g] = accb[g] + wv * b
                    bcol = jnp.full((L,), ph * POOLED + pw, dtype=jnp.int32)
                    for g in range(ng):
                        ch = g * 2 * L + 2 * lanes
                        plsc.store_scatter(outT_v, [ch * BINS + bcol], acca[g])
                        plsc.store_scatter(
                            outT_v, [(ch + 1) * BINS + bcol], accb[g])

                handle = nxt
            pltpu.sync_copy(outT_v, out_hbm.at[roi])

        @pl.loop(0, rpw)
        def _roi_guard(i):
            roi = wid * rpw + i

            @pl.when(roi < N)
            def _roi():
                _roi_body(i, roi)

    return k(flat, params)


def kernel(input, rois):
    B, C, H, W = input.shape
    N = rois.shape[0]
    npad = -(-N // NWORKERS) * NWORKERS
    flat = input.transpose(0, 2, 3, 1).reshape(B * H * W, C).astype(jnp.bfloat16)
    flat = lax.bitcast_convert_type(
        flat.reshape(B * H * W, C // 2, 2), jnp.int32)
    batch = rois[:, 0].astype(jnp.int32)
    base = (batch * (H * W)).astype(jnp.float32)
    cx = rois[:, 1] * SCALE - 0.5
    cy = rois[:, 2] * SCALE - 0.5
    rw = rois[:, 3] * SCALE
    rh = rois[:, 4] * SCALE
    th = rois[:, 5]
    params = jnp.stack(
        [base, cx, cy, rw, rh, jnp.cos(th), jnp.sin(th)], axis=1)
    params = jnp.pad(params, ((0, npad - N), (0, L - params.shape[1])))
    params = params.reshape(npad * L)
    out = _roi_align_sc(flat, params, npad, N, H, W, C)
    return out.reshape(N, C, POOLED, POOLED)


# speedup vs baseline: 7.9869x; 1.2102x over previous
import functools

import jax
import jax.numpy as jnp
from jax import lax
from jax.experimental import pallas as pl
from jax.experimental.pallas import tpu as pltpu
from jax.experimental.pallas import tpu_sc as plsc

SCALE = 0.25
POOLED = 7
BINS = POOLED * POOLED
SAMPLES = BINS * 4
CHUNK = POOLED * 16
NCORES = 2
NSUB = 16
NWORKERS = NCORES * NSUB
L = 16


def _splat(v):
    return jnp.full((L,), v, dtype=jnp.int32)


def _roi_align_sc(flat, params, npad, N, H, W, C):
    rpw = npad // NWORKERS
    ng = C // (2 * L)
    mesh = plsc.VectorSubcoreMesh(
        core_axis_name="c", subcore_axis_name="s",
        num_cores=NCORES, num_subcores=NSUB)

    @functools.partial(
        pl.kernel,
        out_type=jax.ShapeDtypeStruct((N, C * BINS), jnp.float32),
        mesh=mesh,
        scratch_types=[
            pltpu.VMEM((rpw * L,), jnp.float32),
            pltpu.VMEM((POOLED * CHUNK,), jnp.int32),
            pltpu.VMEM((POOLED * CHUNK,), jnp.float32),
            pltpu.VMEM((2, CHUNK, C // 2), jnp.int32),
            pltpu.VMEM((C * BINS,), jnp.float32),
            pltpu.SemaphoreType.DMA,
            pltpu.SemaphoreType.DMA,
        ],
        compiler_params=pltpu.CompilerParams(needs_layout_passes=False),
    )
    def k(flat_hbm, params_hbm, out_hbm,
          params_v, idx_v, w_v, rows_v, outT_v, sem0, sem1):
        sems = (sem0, sem1)
        wid = lax.axis_index("s") * NCORES + lax.axis_index("c")
        pltpu.sync_copy(params_hbm.at[pl.ds(wid * (rpw * L), rpw * L)], params_v)
        lanes = lax.iota(jnp.int32, L)

        def _roi_body(i, roi):
            iv = jnp.full((L,), i, dtype=jnp.int32)

            def pget(j):
                return plsc.load_gather(params_v, [iv * L + _splat(j)])

            base_i = pget(0).astype(jnp.int32)
            cxv, cyv = pget(1), pget(2)
            rwv, rhv = pget(3), pget(4)
            csv, snv = pget(5), pget(6)

            for j in range((SAMPLES + L - 1) // L):
                s = lanes + (L * j)
                ph = lax.div(s, _splat(28))
                r = lax.rem(s, _splat(28))
                pw = lax.div(r, _splat(4))
                q = lax.rem(r, _splat(4))
                iy = lax.div(q, _splat(2))
                ix = lax.rem(q, _splat(2))
                fy = (ph.astype(jnp.float32)
                      + (iy.astype(jnp.float32) * 0.5 + 0.25)) * (1.0 / POOLED) - 0.5
                fx = (pw.astype(jnp.float32)
                      + (ix.astype(jnp.float32) * 0.5 + 0.25)) * (1.0 / POOLED) - 0.5
                yy = rhv * fy
                xx = rwv * fx
                y = yy * csv - xx * snv + cyv
                x = yy * snv + xx * csv + cxv
                valid = (y > -1.0) & (y < H) & (x > -1.0) & (x < W)
                yc = jnp.maximum(y, 0.0)
                xc = jnp.maximum(x, 0.0)
                yl = yc.astype(jnp.int32)
                xl = xc.astype(jnp.int32)
                ycond = yl >= H - 1
                xcond = xl >= W - 1
                yl = jnp.where(ycond, H - 1, yl)
                xl = jnp.where(xcond, W - 1, xl)
                yh = jnp.where(ycond, H - 1, yl + 1)
                xh = jnp.where(xcond, W - 1, xl + 1)
                ly = jnp.where(ycond, 0.0, yc - yl.astype(jnp.float32))
                lx = jnp.where(xcond, 0.0, xc - xl.astype(jnp.float32))
                hy = 1.0 - ly
                hx = 1.0 - lx
                vm = jnp.where(valid, 0.25, 0.0)
                rl = base_i + yl * W
                rh_ = base_i + yh * W
                col = s * 4
                mask = s < SAMPLES
                entries = (
                    (rl + xl, hy * hx * vm),
                    (rl + xh, hy * lx * vm),
                    (rh_ + xl, ly * hx * vm),
                    (rh_ + xh, ly * lx * vm),
                )
                for c, (ivec, wvec) in enumerate(entries):
                    plsc.store_scatter(idx_v, [col + c], ivec, mask=mask)
                    plsc.store_scatter(w_v, [col + c], wvec, mask=mask)

            def start(ph):
                slot = ph % 2
                return pltpu.async_copy(
                    flat_hbm.at[idx_v.at[pl.ds(ph * CHUNK, CHUNK)]],
                    rows_v.at[slot], sems[slot])

            handle = start(0)
            for ph in range(POOLED):
                nxt = start(ph + 1) if ph + 1 < POOLED else None
                handle.wait()
                slot = ph % 2

                @pl.loop(0, POOLED)
                def _bin(pw, ph=ph, slot=slot):
                    acca = [None] * ng
                    accb = [None] * ng
                    for kk in range(16):
                        wv = plsc.load_gather(
                            w_v, [jnp.full((L,), ph * CHUNK + pw * 16 + kk,
                                           jnp.int32)])
                        row = pw * 16 + kk
                        for g in range(ng):
                            blk = plsc.bitcast(
                                rows_v[slot, row, pl.ds(g * L, L)], jnp.bfloat16)
                            a, b = plsc.unpack(
                                blk, format=plsc.PackFormat.INTERLEAVED)
                            if kk == 0:
                                acca[g] = wv * a
                                accb[g] = wv * b
                            else:
                                acca[g] = acca[g] + wv * a
                                accb[g] = accb[g] + wv * b
                    bcol = jnp.full((L,), ph * POOLED + pw, dtype=jnp.int32)
                    for g in range(ng):
                        ch = g * 2 * L + 2 * lanes
                        plsc.store_scatter(outT_v, [ch * BINS + bcol], acca[g])
                        plsc.store_scatter(
                            outT_v, [(ch + 1) * BINS + bcol], accb[g])

                handle = nxt
            pltpu.sync_copy(outT_v, out_hbm.at[roi])

        @pl.loop(0, rpw)
        def _roi_guard(i):
            roi = wid * rpw + i

            @pl.when(roi < N)
            def _roi():
                _roi_body(i, roi)

    return k(flat, params)


_HR = 8


def _prep_kernel(x_ref, o_ref):
    v = x_ref[0]
    u = lax.bitcast_convert_type(
        v.astype(jnp.bfloat16), jnp.uint16).astype(jnp.uint32)
    u2 = u.reshape(u.shape[0] // 2, 2, _HR, u.shape[2])
    w = ((u2[:, 1] << 16) | u2[:, 0]).astype(jnp.int32)
    W = v.shape[2]
    for hh in range(_HR):
        o_ref[0, hh * W:(hh + 1) * W, :] = jnp.swapaxes(w[:, hh, :], 0, 1)


def _prep(x):
    B, C, H, W = x.shape
    g = H // _HR
    out = pl.pallas_call(
        _prep_kernel,
        grid=(B * g,),
        in_specs=[pl.BlockSpec(
            (1, C, _HR, W), lambda i: (i // g, 0, i % g, 0))],
        out_specs=pl.BlockSpec((1, _HR * W, C // 2), lambda i: (i, 0, 0)),
        out_shape=jax.ShapeDtypeStruct((B * g, _HR * W, C // 2), jnp.int32),
    )(x)
    return out.reshape(B * H * W, C // 2)


def kernel(input, rois):
    B, C, H, W = input.shape
    N = rois.shape[0]
    npad = -(-N // NWORKERS) * NWORKERS
    flat = _prep(input)
    batch = rois[:, 0].astype(jnp.int32)
    base = (batch * (H * W)).astype(jnp.float32)
    cx = rois[:, 1] * SCALE - 0.5
    cy = rois[:, 2] * SCALE - 0.5
    rw = rois[:, 3] * SCALE
    rh = rois[:, 4] * SCALE
    th = rois[:, 5]
    params = jnp.stack(
        [base, cx, cy, rw, rh, jnp.cos(th), jnp.sin(th)], axis=1)
    params = jnp.pad(params, ((0, npad - N), (0, L - params.shape[1])))
    params = params.reshape(npad * L)
    out = _roi_align_sc(flat, params, npad, N, H, W, C)
    return out.reshape(N, C, POOLED, POOLED)
